# baseline (speedup 1.0000x reference)
import functools
import jax
import jax.numpy as jnp
from jax import lax
from jax.experimental import pallas as pl
from jax.experimental.pallas import tpu as pltpu
from jax.experimental.pallas import tpu_sc as plsc

N = 10000
E = 160000
NUM_RBF = 32
UNITS = 128
CUTOFF = 5.0

BE = 640
BN = 200
BNC = 80

NW = 32
NPW = 320
IPW = 328
IPV = 344
KE = 24
EPAD = -(-E // KE) * KE
NBMAX = EPAD // KE
PED = 10 * UNITS
NCH = PED // 16


def _mlp_body(ea_ref, ew_ref, w1_ref, b1_ref, w2_ref, b2_ref, w3_ref, b3_ref,
              out_ref):
    a = ea_ref[...]
    h = jax.nn.silu(jnp.dot(a, w1_ref[...], preferred_element_type=jnp.float32)
                    + b1_ref[...])
    h = jax.nn.silu(jnp.dot(h, w2_ref[...], preferred_element_type=jnp.float32)
                    + b2_ref[...])
    h = jax.nn.silu(jnp.dot(h, w3_ref[...], preferred_element_type=jnp.float32)
                    + b3_ref[...])
    r = ew_ref[...]
    c = 0.5 * (jnp.cos(jnp.pi * r / CUTOFF) + 1.0) * (r < CUTOFF).astype(r.dtype)
    out_ref[...] = h * c


def _run_mlp(ea, ew, W_s1T, b_s1, W_s2T, b_s2, W_s3rT, b_s3r):
    grid = E // BE
    full = lambda shape: pl.BlockSpec(shape, lambda i: (0,) * len(shape))
    return pl.pallas_call(
        _mlp_body,
        grid=(grid,),
        in_specs=[
            pl.BlockSpec((BE, NUM_RBF), lambda i: (i, 0)),
            pl.BlockSpec((BE, 1), lambda i: (i, 0)),
            full((NUM_RBF, UNITS)), full((1, UNITS)),
            full((UNITS, 2 * UNITS)), full((1, 2 * UNITS)),
            full((2 * UNITS, 3 * UNITS)), full((1, 3 * UNITS)),
        ],
        out_specs=pl.BlockSpec((BE, 3 * UNITS), lambda i: (i, 0)),
        out_shape=jax.ShapeDtypeStruct((E, 3 * UNITS), jnp.float32),
    )(ea, ew, W_s1T, b_s1, W_s2T, b_s2, W_s3rT, b_s3r)


def _normed_components(x_ref):
    x = [[x_ref[:, i, j, :] for j in range(3)] for i in range(3)]
    nrm = sum(x[i][j] * x[i][j] for i in range(3) for j in range(3)) + 1.0
    inv = 1.0 / nrm
    xn = [[x[i][j] * inv for j in range(3)] for i in range(3)]
    return xn


def _pack_body(x_ref, w0_ref, w1_ref, w2_ref, out_ref):
    xn = _normed_components(x_ref)
    t = (xn[0][0] + xn[1][1] + xn[2][2]) * (1.0 / 3.0)
    a01 = 0.5 * (xn[0][1] - xn[1][0])
    a02 = 0.5 * (xn[0][2] - xn[2][0])
    a12 = 0.5 * (xn[1][2] - xn[2][1])
    s00 = xn[0][0] - t
    s11 = xn[1][1] - t
    s22 = xn[2][2] - t
    s01 = 0.5 * (xn[0][1] + xn[1][0])
    s02 = 0.5 * (xn[0][2] + xn[2][0])
    s12 = 0.5 * (xn[1][2] + xn[2][1])
    p0 = jnp.dot(t, w0_ref[...], preferred_element_type=jnp.float32)
    pa = jnp.dot(jnp.concatenate([a01, a02, a12], axis=0), w1_ref[...],
                 preferred_element_type=jnp.float32)
    ps = jnp.dot(jnp.concatenate([s00, s01, s02, s11, s12, s22], axis=0),
                 w2_ref[...], preferred_element_type=jnp.float32)
    b = p0.shape[0]
    out_ref[:, 0, :] = p0
    for k in range(3):
        out_ref[:, 1 + k, :] = pa[k * b:(k + 1) * b]
    for k in range(6):
        out_ref[:, 4 + k, :] = ps[k * b:(k + 1) * b]


def _run_pack(X, W_t0T, W_t1T, W_t2T):
    grid = N // BN
    full = lambda shape: pl.BlockSpec(shape, lambda i: (0,) * len(shape))
    return pl.pallas_call(
        _pack_body,
        grid=(grid,),
        in_specs=[
            pl.BlockSpec((BN, 3, 3, UNITS), lambda i: (i, 0, 0, 0)),
            full((UNITS, UNITS)), full((UNITS, UNITS)), full((UNITS, UNITS)),
        ],
        out_specs=pl.BlockSpec((BN, 10, UNITS), lambda i: (i, 0, 0)),
        out_shape=jax.ShapeDtypeStruct((N, 10, UNITS), jnp.float32),
    )(X, W_t0T, W_t1T, W_t2T)


def _tensor_from_rows(p):
    t, a01, a02, a12, s00, s01, s02, s11, s12, s22 = p
    return [
        [t + s00, a01 + s01, a02 + s02],
        [s01 - a01, t + s11, a12 + s12],
        [s02 - a02, s12 - a12, t + s22],
    ]


def _post_body(x_ref, p_ref, mp_ref, w3_ref, w4_ref, w5_ref, out_ref):
    xn = _normed_components(x_ref)
    Y = _tensor_from_rows([p_ref[:, k, :] for k in range(10)])
    M = _tensor_from_rows([mp_ref[:, k, :] for k in range(10)])
    Cm = [[sum(Y[i][j] * M[j][k] + M[i][j] * Y[j][k] for j in range(3))
           for k in range(3)] for i in range(3)]
    nrm = sum(Cm[i][j] * Cm[i][j] for i in range(3) for j in range(3)) + 1.0
    inv = 1.0 / nrm
    c = [[Cm[i][j] * inv for j in range(3)] for i in range(3)]
    t2 = (c[0][0] + c[1][1] + c[2][2]) * (1.0 / 3.0)
    a2 = [0.5 * (c[0][1] - c[1][0]), 0.5 * (c[0][2] - c[2][0]),
          0.5 * (c[1][2] - c[2][1])]
    s2 = [c[0][0] - t2, 0.5 * (c[0][1] + c[1][0]), 0.5 * (c[0][2] + c[2][0]),
          c[1][1] - t2, 0.5 * (c[1][2] + c[2][1]), c[2][2] - t2]
    b = t2.shape[0]
    dt = jnp.dot(t2, w3_ref[...], preferred_element_type=jnp.float32)
    da = jnp.dot(jnp.concatenate(a2, axis=0), w4_ref[...],
                 preferred_element_type=jnp.float32)
    ds = jnp.dot(jnp.concatenate(s2, axis=0), w5_ref[...],
                 preferred_element_type=jnp.float32)
    rows = [dt, da[0:b], da[b:2 * b], da[2 * b:3 * b],
            ds[0:b], ds[b:2 * b], ds[2 * b:3 * b], ds[3 * b:4 * b],
            ds[4 * b:5 * b], ds[5 * b:6 * b]]
    dX = _tensor_from_rows(rows)
    for i in range(3):
        for k in range(3):
            out_ref[:, i, k, :] = (xn[i][k] + dX[i][k]
                                   + sum(dX[i][j] * dX[j][k] for j in range(3)))


def _run_post(X, P, MP, W_t3T, W_t4T, W_t5T):
    grid = N // BNC
    full = lambda shape: pl.BlockSpec(shape, lambda i: (0,) * len(shape))
    return pl.pallas_call(
        _post_body,
        grid=(grid,),
        in_specs=[
            pl.BlockSpec((BNC, 3, 3, UNITS), lambda i: (i, 0, 0, 0)),
            pl.BlockSpec((BNC, 10, UNITS), lambda i: (i, 0, 0)),
            pl.BlockSpec((BNC, 10, UNITS), lambda i: (i, 0, 0)),
            full((UNITS, UNITS)), full((UNITS, UNITS)), full((UNITS, UNITS)),
        ],
        out_specs=pl.BlockSpec((BNC, 3, 3, UNITS), lambda i: (i, 0, 0, 0)),
        out_shape=jax.ShapeDtypeStruct((N, 3, 3, UNITS), jnp.float32),
    )(X, P, MP, W_t3T, W_t4T, W_t5T)


def _sc_mesh():
    return plsc.VectorSubcoreMesh(core_axis_name="c", subcore_axis_name="s")


_W_GROUP = [0, 1, 1, 1, 2, 2, 2, 2, 2, 2]


def _run_msg(Pflat, w, row_indices, row_data, iptr_pad):
    @functools.partial(
        pl.kernel,
        out_type=jax.ShapeDtypeStruct((N, PED), jnp.float32),
        mesh=_sc_mesh(),
        scratch_types=[
            pltpu.VMEM((IPV,), jnp.int32),
            pltpu.VMEM((KE,), jnp.int32),
            pltpu.VMEM((KE,), jnp.int32),
            pltpu.VMEM((KE,), jnp.int32),
            pltpu.VMEM((KE,), jnp.int32),
            pltpu.VMEM((KE, PED), jnp.float32),
            pltpu.VMEM((KE, PED), jnp.float32),
            pltpu.VMEM((KE, 3 * UNITS), jnp.float32),
            pltpu.VMEM((KE, 3 * UNITS), jnp.float32),
            pltpu.VMEM((KE, PED), jnp.float32),
            pltpu.VMEM((PED,), jnp.float32),
            pltpu.SMEM((KE,), jnp.int32),
            pltpu.SemaphoreType.DMA,
            pltpu.SemaphoreType.DMA,
            pltpu.SemaphoreType.DMA,
            pltpu.SemaphoreType.DMA,
            pltpu.SemaphoreType.DMA,
            pltpu.SemaphoreType.DMA,
            pltpu.SemaphoreType.DMA,
        ],
    )
    def k(p_hbm, w_hbm, ri_hbm, rd_hbm, ip_hbm, out_hbm,
          iptr_v, idx0, idx1, wdx0, wdx1, pb0, pb1, wb0, wb1, ring, acc, ids,
          si0, si1, sg0, sg1, sw0, sw1, sf):
        wid = lax.axis_index("c") * 16 + lax.axis_index("s")
        n0 = wid * NPW
        pltpu.sync_copy(ip_hbm.at[pl.ds(n0, IPW)], iptr_v.at[pl.ds(0, IPW)])

        def rd(i):
            return iptr_v[pl.ds(i, 16)][0]

        zvec = jnp.zeros((16,), jnp.float32)

        def zero_acc():
            for c in range(NCH):
                acc[pl.ds(16 * c, 16)] = zvec

        zero_acc()

        e0 = rd(0)
        e1 = rd(NPW)
        nb = (e1 + KE - 1) // KE - e0 // KE
        b0 = e0 // KE

        def binsearch(e):
            lo = jnp.int32(0)
            for s in (256, 128, 64, 32, 16, 8, 4, 2, 1):
                cand = lo + s
                ok = jnp.logical_and(cand <= NPW - 1, rd(cand) <= e)
                lo = jnp.where(ok, cand, lo)
            return lo

        l_init = binsearch(e0)

        bufs = ((idx0, wdx0, pb0, wb0, si0, sg0, sw0),
                (idx1, wdx1, pb1, wb1, si1, sg1, sw1))

        def idx_fetch(r, t):
            ix, wx, _, _, si, _, _ = bufs[t]
            b = b0 + r
            pltpu.async_copy(ri_hbm.at[pl.ds(b * KE, KE)], ix, si)
            pltpu.async_copy(rd_hbm.at[pl.ds(b * KE, KE)], wx, si)

        def idx_wait(r, t):
            ix, wx, _, _, si, _, _ = bufs[t]
            b = b0 + r
            pltpu.make_async_copy(ri_hbm.at[pl.ds(b * KE, KE)], ix, si).wait()
            pltpu.make_async_copy(rd_hbm.at[pl.ds(b * KE, KE)], wx, si).wait()

        def data_fetch(t):
            ix, wx, pb, wb, _, sg, sw = bufs[t]
            pltpu.async_copy(p_hbm.at[ix], pb, sg)
            pltpu.async_copy(w_hbm.at[wx], wb, sw)

        def data_wait(t):
            ix, wx, pb, wb, _, sg, sw = bufs[t]
            pltpu.make_async_copy(p_hbm.at[ix], pb, sg).wait()
            pltpu.make_async_copy(w_hbm.at[wx], wb, sw).wait()

        @pl.when(nb > 0)
        def _():
            b = b0
            pltpu.sync_copy(ri_hbm.at[pl.ds(b * KE, KE)], idx0)
            pltpu.sync_copy(rd_hbm.at[pl.ds(b * KE, KE)], wdx0)
            data_fetch(0)

        @pl.when(nb > 1)
        def _():
            idx_fetch(1, 1)

        def do_block(r, t, carry):
            l, rcp = carry
            b = b0 + r

            @pl.when(r + 1 < nb)
            def _():
                idx_wait(r + 1, 1 - t)
                data_fetch(1 - t)

            @pl.when(r < nb)
            def _():
                data_wait(t)

            @pl.when(r + 2 < nb)
            def _():
                idx_fetch(r + 2, t)

            def fwait(s, _):
                @pl.when(s < rcp)
                def _():
                    pltpu.make_async_copy(ring.at[s], out_hbm.at[0], sf).wait()
                return 0

            lax.fori_loop(0, KE, fwait, 0)

            _, _, pb, wb, _, _, _ = bufs[t]
            jlo = jnp.maximum(e0 - b * KE, 0)
            jhi = jnp.minimum(e1 - b * KE, KE)

            def jbody(j, c2):
                l_, rc = c2
                e = b * KE + j
                close = rd(l_ + 1) <= e

                @pl.when(close)
                def _():
                    for c3 in range(NCH):
                        sl = pl.ds(16 * c3, 16)
                        ring[rc, sl] = acc[sl]
                        acc[sl] = zvec
                    ids[rc] = n0 + l_

                rc = rc + close.astype(jnp.int32)
                l_ = jnp.where(close, binsearch(e), l_)

                wv = [wb[j, pl.ds(128 * g + 16 * m, 16)]
                      for g in range(3) for m in range(8)]
                for k10 in range(10):
                    g = _W_GROUP[k10]
                    for m in range(8):
                        ch = k10 * 8 + m
                        plsc.addupdate(
                            acc.at[pl.ds(16 * ch, 16)],
                            pb[j, pl.ds(16 * ch, 16)] * wv[g * 8 + m])
                return (l_, rc)

            l, rc = lax.fori_loop(jlo, jhi, jbody, (l, jnp.int32(0)))

            def fissue(s, _):
                @pl.when(s < rc)
                def _():
                    pltpu.async_copy(ring.at[s], out_hbm.at[ids[s]], sf)
                return 0

            lax.fori_loop(0, KE, fissue, 0)

            return (l, rc)

        def pair_body(i, carry):
            carry = do_block(2 * i, 0, carry)
            carry = do_block(2 * i + 1, 1, carry)
            return carry

        l, rcp = lax.fori_loop(0, (NBMAX + 1) // 2, pair_body,
                               (l_init, jnp.int32(0)))

        def fwait2(s, _):
            @pl.when(s < rcp)
            def _():
                pltpu.make_async_copy(ring.at[s], out_hbm.at[0], sf).wait()
            return 0

        lax.fori_loop(0, KE, fwait2, 0)

        @pl.when(e1 > e0)
        def _():
            pltpu.sync_copy(acc, out_hbm.at[n0 + l])

        for c in range(NCH):
            ring[0, pl.ds(16 * c, 16)] = zvec

        def zbody(z, _):
            @pl.when(jnp.logical_and(n0 + z < N, rd(z) == rd(z + 1)))
            def _():
                pltpu.sync_copy(ring.at[0], out_hbm.at[n0 + z])
            return 0

        lax.fori_loop(0, NPW, zbody, 0)

    return k(Pflat, w, row_indices, row_data, iptr_pad)


def kernel(X, edge_index, edge_weight, edge_attr, row_data, row_indices,
           row_indptr, col_data, col_indices, col_indptr,
           W_s1, b_s1, W_s2, b_s2, W_s3, b_s3,
           W_t0, W_t1, W_t2, W_t3, W_t4, W_t5):
    perm = jnp.arange(3 * UNITS).reshape(UNITS, 3).T.reshape(-1)
    W_s3r = W_s3[perm]
    b_s3r = b_s3[perm]

    w = _run_mlp(edge_attr, edge_weight[:, None], W_s1.T, b_s1[None],
                 W_s2.T, b_s2[None],
                 W_s3r.T, b_s3r[None])

    P = _run_pack(X, W_t0.T, W_t1.T, W_t2.T)

    iptr_pad = jnp.concatenate(
        [row_indptr, jnp.full((IPW + NPW,), E, jnp.int32)])
    ri_pad = jnp.concatenate(
        [row_indices, jnp.zeros((EPAD - E,), jnp.int32)])
    rdt_pad = jnp.concatenate(
        [row_data, jnp.zeros((EPAD - E,), jnp.int32)])
    MP = _run_msg(P.reshape(N, PED), w, ri_pad, rdt_pad,
                  iptr_pad).reshape(N, 10, UNITS)

    return _run_post(X, P, MP, W_t3.T, W_t4.T, W_t5.T)


# speedup vs baseline: 1.5809x; 1.5809x over previous
import functools
import jax
import jax.numpy as jnp
from jax import lax
from jax.experimental import pallas as pl
from jax.experimental.pallas import tpu as pltpu
from jax.experimental.pallas import tpu_sc as plsc

N = 10000
E = 160000
NUM_RBF = 32
UNITS = 128
CUTOFF = 5.0

BE = 640
BN = 200
BNC = 80

NW = 32
NPW = 320
IPW = 328
IPV = 344
KE = 24
EPAD = -(-E // KE) * KE
NBMAX = EPAD // KE
PED = 10 * UNITS
NCH = PED // 16


def _mlp_body(ea_ref, ew_ref, w1_ref, b1_ref, w2_ref, b2_ref, w3_ref, b3_ref,
              out_ref):
    a = ea_ref[...]
    h = jax.nn.silu(jnp.dot(a, w1_ref[...], preferred_element_type=jnp.float32)
                    + b1_ref[...])
    h = jax.nn.silu(jnp.dot(h, w2_ref[...], preferred_element_type=jnp.float32)
                    + b2_ref[...])
    h = jax.nn.silu(jnp.dot(h, w3_ref[...], preferred_element_type=jnp.float32)
                    + b3_ref[...])
    r = ew_ref[...]
    c = 0.5 * (jnp.cos(jnp.pi * r / CUTOFF) + 1.0) * (r < CUTOFF).astype(r.dtype)
    out_ref[...] = h * c


def _run_mlp(ea, ew, W_s1T, b_s1, W_s2T, b_s2, W_s3rT, b_s3r):
    grid = E // BE
    full = lambda shape: pl.BlockSpec(shape, lambda i: (0,) * len(shape))
    return pl.pallas_call(
        _mlp_body,
        grid=(grid,),
        in_specs=[
            pl.BlockSpec((BE, NUM_RBF), lambda i: (i, 0)),
            pl.BlockSpec((BE, 1), lambda i: (i, 0)),
            full((NUM_RBF, UNITS)), full((1, UNITS)),
            full((UNITS, 2 * UNITS)), full((1, 2 * UNITS)),
            full((2 * UNITS, 3 * UNITS)), full((1, 3 * UNITS)),
        ],
        out_specs=pl.BlockSpec((BE, 3 * UNITS), lambda i: (i, 0)),
        out_shape=jax.ShapeDtypeStruct((E, 3 * UNITS), jnp.float32),
    )(ea, ew, W_s1T, b_s1, W_s2T, b_s2, W_s3rT, b_s3r)


def _normed_components(x_ref):
    x = [[x_ref[:, i, j, :] for j in range(3)] for i in range(3)]
    nrm = sum(x[i][j] * x[i][j] for i in range(3) for j in range(3)) + 1.0
    inv = 1.0 / nrm
    xn = [[x[i][j] * inv for j in range(3)] for i in range(3)]
    return xn


def _pack_body(x_ref, w0_ref, w1_ref, w2_ref, out_ref):
    xn = _normed_components(x_ref)
    t = (xn[0][0] + xn[1][1] + xn[2][2]) * (1.0 / 3.0)
    a01 = 0.5 * (xn[0][1] - xn[1][0])
    a02 = 0.5 * (xn[0][2] - xn[2][0])
    a12 = 0.5 * (xn[1][2] - xn[2][1])
    s00 = xn[0][0] - t
    s11 = xn[1][1] - t
    s22 = xn[2][2] - t
    s01 = 0.5 * (xn[0][1] + xn[1][0])
    s02 = 0.5 * (xn[0][2] + xn[2][0])
    s12 = 0.5 * (xn[1][2] + xn[2][1])
    p0 = jnp.dot(t, w0_ref[...], preferred_element_type=jnp.float32)
    pa = jnp.dot(jnp.concatenate([a01, a02, a12], axis=0), w1_ref[...],
                 preferred_element_type=jnp.float32)
    ps = jnp.dot(jnp.concatenate([s00, s01, s02, s11, s12, s22], axis=0),
                 w2_ref[...], preferred_element_type=jnp.float32)
    b = p0.shape[0]
    out_ref[:, 0, :] = p0
    for k in range(3):
        out_ref[:, 1 + k, :] = pa[k * b:(k + 1) * b]
    for k in range(6):
        out_ref[:, 4 + k, :] = ps[k * b:(k + 1) * b]


def _run_pack(X, W_t0T, W_t1T, W_t2T):
    grid = N // BN
    full = lambda shape: pl.BlockSpec(shape, lambda i: (0,) * len(shape))
    return pl.pallas_call(
        _pack_body,
        grid=(grid,),
        in_specs=[
            pl.BlockSpec((BN, 3, 3, UNITS), lambda i: (i, 0, 0, 0)),
            full((UNITS, UNITS)), full((UNITS, UNITS)), full((UNITS, UNITS)),
        ],
        out_specs=pl.BlockSpec((BN, 10, UNITS), lambda i: (i, 0, 0)),
        out_shape=jax.ShapeDtypeStruct((N, 10, UNITS), jnp.float32),
    )(X, W_t0T, W_t1T, W_t2T)


def _tensor_from_rows(p):
    t, a01, a02, a12, s00, s01, s02, s11, s12, s22 = p
    return [
        [t + s00, a01 + s01, a02 + s02],
        [s01 - a01, t + s11, a12 + s12],
        [s02 - a02, s12 - a12, t + s22],
    ]


def _post_body(x_ref, p_ref, mp_ref, w3_ref, w4_ref, w5_ref, out_ref):
    xn = _normed_components(x_ref)
    Y = _tensor_from_rows([p_ref[:, k, :] for k in range(10)])
    M = _tensor_from_rows([mp_ref[:, k, :] for k in range(10)])
    Cm = [[sum(Y[i][j] * M[j][k] + M[i][j] * Y[j][k] for j in range(3))
           for k in range(3)] for i in range(3)]
    nrm = sum(Cm[i][j] * Cm[i][j] for i in range(3) for j in range(3)) + 1.0
    inv = 1.0 / nrm
    c = [[Cm[i][j] * inv for j in range(3)] for i in range(3)]
    t2 = (c[0][0] + c[1][1] + c[2][2]) * (1.0 / 3.0)
    a2 = [0.5 * (c[0][1] - c[1][0]), 0.5 * (c[0][2] - c[2][0]),
          0.5 * (c[1][2] - c[2][1])]
    s2 = [c[0][0] - t2, 0.5 * (c[0][1] + c[1][0]), 0.5 * (c[0][2] + c[2][0]),
          c[1][1] - t2, 0.5 * (c[1][2] + c[2][1]), c[2][2] - t2]
    b = t2.shape[0]
    dt = jnp.dot(t2, w3_ref[...], preferred_element_type=jnp.float32)
    da = jnp.dot(jnp.concatenate(a2, axis=0), w4_ref[...],
                 preferred_element_type=jnp.float32)
    ds = jnp.dot(jnp.concatenate(s2, axis=0), w5_ref[...],
                 preferred_element_type=jnp.float32)
    rows = [dt, da[0:b], da[b:2 * b], da[2 * b:3 * b],
            ds[0:b], ds[b:2 * b], ds[2 * b:3 * b], ds[3 * b:4 * b],
            ds[4 * b:5 * b], ds[5 * b:6 * b]]
    dX = _tensor_from_rows(rows)
    for i in range(3):
        for k in range(3):
            out_ref[:, i, k, :] = (xn[i][k] + dX[i][k]
                                   + sum(dX[i][j] * dX[j][k] for j in range(3)))


def _run_post(X, P, MP, W_t3T, W_t4T, W_t5T):
    grid = N // BNC
    full = lambda shape: pl.BlockSpec(shape, lambda i: (0,) * len(shape))
    return pl.pallas_call(
        _post_body,
        grid=(grid,),
        in_specs=[
            pl.BlockSpec((BNC, 3, 3, UNITS), lambda i: (i, 0, 0, 0)),
            pl.BlockSpec((BNC, 10, UNITS), lambda i: (i, 0, 0)),
            pl.BlockSpec((BNC, 10, UNITS), lambda i: (i, 0, 0)),
            full((UNITS, UNITS)), full((UNITS, UNITS)), full((UNITS, UNITS)),
        ],
        out_specs=pl.BlockSpec((BNC, 3, 3, UNITS), lambda i: (i, 0, 0, 0)),
        out_shape=jax.ShapeDtypeStruct((N, 3, 3, UNITS), jnp.float32),
    )(X, P, MP, W_t3T, W_t4T, W_t5T)


def _sc_mesh():
    return plsc.VectorSubcoreMesh(core_axis_name="c", subcore_axis_name="s")


_W_GROUP = [0, 1, 1, 1, 2, 2, 2, 2, 2, 2]


def _run_msg(Pflat, w, row_indices, row_data, iptr_pad):
    @functools.partial(
        pl.kernel,
        out_type=jax.ShapeDtypeStruct((N, PED), jnp.float32),
        mesh=_sc_mesh(),
        scratch_types=[
            pltpu.VMEM((IPV,), jnp.int32),
            pltpu.VMEM((KE,), jnp.int32),
            pltpu.VMEM((KE,), jnp.int32),
            pltpu.VMEM((KE,), jnp.int32),
            pltpu.VMEM((KE,), jnp.int32),
            pltpu.VMEM((KE, PED), jnp.float32),
            pltpu.VMEM((KE, PED), jnp.float32),
            pltpu.VMEM((KE, 3 * UNITS), jnp.float32),
            pltpu.VMEM((KE, 3 * UNITS), jnp.float32),
            pltpu.VMEM((KE, PED), jnp.float32),
            pltpu.VMEM((PED,), jnp.float32),
            pltpu.SMEM((KE,), jnp.int32),
            pltpu.SemaphoreType.DMA,
            pltpu.SemaphoreType.DMA,
            pltpu.SemaphoreType.DMA,
            pltpu.SemaphoreType.DMA,
            pltpu.SemaphoreType.DMA,
            pltpu.SemaphoreType.DMA,
            pltpu.SemaphoreType.DMA,
        ],
    )
    def k(p_hbm, w_hbm, ri_hbm, rd_hbm, ip_hbm, out_hbm,
          iptr_v, idx0, idx1, wdx0, wdx1, pb0, pb1, wb0, wb1, ring, acc, ids,
          si0, si1, sg0, sg1, sw0, sw1, sf):
        wid = lax.axis_index("c") * 16 + lax.axis_index("s")
        n0 = wid * NPW
        pltpu.sync_copy(ip_hbm.at[pl.ds(n0, IPW)], iptr_v.at[pl.ds(0, IPW)])

        def rd(i):
            return iptr_v[pl.ds(i, 16)][0]

        zvec = jnp.zeros((16,), jnp.float32)

        def zero_acc():
            for c in range(NCH):
                acc[pl.ds(16 * c, 16)] = zvec

        zero_acc()

        e0 = rd(0)
        e1 = rd(NPW)
        nb = (e1 + KE - 1) // KE - e0 // KE
        b0 = e0 // KE

        def binsearch(e):
            lo = jnp.int32(0)
            for s in (256, 128, 64, 32, 16, 8, 4, 2, 1):
                cand = lo + s
                ok = jnp.logical_and(cand <= NPW - 1, rd(cand) <= e)
                lo = jnp.where(ok, cand, lo)
            return lo

        l_init = binsearch(e0)

        bufs = ((idx0, wdx0, pb0, wb0, si0, sg0, sw0),
                (idx1, wdx1, pb1, wb1, si1, sg1, sw1))

        def idx_fetch(r, t):
            ix, wx, _, _, si, _, _ = bufs[t]
            b = b0 + r
            pltpu.async_copy(ri_hbm.at[pl.ds(b * KE, KE)], ix, si)
            pltpu.async_copy(rd_hbm.at[pl.ds(b * KE, KE)], wx, si)

        def idx_wait(r, t):
            ix, wx, _, _, si, _, _ = bufs[t]
            b = b0 + r
            pltpu.make_async_copy(ri_hbm.at[pl.ds(b * KE, KE)], ix, si).wait()
            pltpu.make_async_copy(rd_hbm.at[pl.ds(b * KE, KE)], wx, si).wait()

        def data_fetch(t):
            ix, wx, pb, wb, _, sg, sw = bufs[t]
            pltpu.async_copy(p_hbm.at[ix], pb, sg)
            pltpu.async_copy(w_hbm.at[wx], wb, sw)

        def data_wait(t):
            ix, wx, pb, wb, _, sg, sw = bufs[t]
            pltpu.make_async_copy(p_hbm.at[ix], pb, sg).wait()
            pltpu.make_async_copy(w_hbm.at[wx], wb, sw).wait()

        @pl.when(nb > 0)
        def _():
            b = b0
            pltpu.sync_copy(ri_hbm.at[pl.ds(b * KE, KE)], idx0)
            pltpu.sync_copy(rd_hbm.at[pl.ds(b * KE, KE)], wdx0)
            data_fetch(0)

        @pl.when(nb > 1)
        def _():
            idx_fetch(1, 1)

        def do_block(r, t, carry):
            l, rcp = carry
            b = b0 + r

            @pl.when(r + 1 < nb)
            def _():
                idx_wait(r + 1, 1 - t)
                data_fetch(1 - t)

            @pl.when(r < nb)
            def _():
                data_wait(t)

            @pl.when(r + 2 < nb)
            def _():
                idx_fetch(r + 2, t)

            @pl.when(rcp > 0)
            def _():
                def fwait(s, _):
                    @pl.when(s < rcp)
                    def _():
                        pltpu.make_async_copy(ring.at[s], out_hbm.at[0],
                                              sf).wait()
                    return 0

                lax.fori_loop(0, KE, fwait, 0)

            _, _, pb, wb, _, _, _ = bufs[t]
            jlo = jnp.maximum(e0 - b * KE, 0)
            jhi = jnp.minimum(e1 - b * KE, KE)

            def jbody(j, c2):
                l_, rc = c2
                e = b * KE + j
                close = rd(l_ + 1) <= e

                @pl.when(close)
                def _():
                    for c3 in range(NCH):
                        sl = pl.ds(16 * c3, 16)
                        ring[rc, sl] = acc[sl]
                        acc[sl] = zvec
                    ids[rc] = n0 + l_

                rc = rc + close.astype(jnp.int32)
                l_ = jnp.where(close, binsearch(e), l_)

                wv = [wb[j, pl.ds(128 * g + 16 * m, 16)]
                      for g in range(3) for m in range(8)]
                for k10 in range(10):
                    g = _W_GROUP[k10]
                    for m in range(8):
                        ch = k10 * 8 + m
                        plsc.addupdate(
                            acc.at[pl.ds(16 * ch, 16)],
                            pb[j, pl.ds(16 * ch, 16)] * wv[g * 8 + m])
                return (l_, rc)

            l, rc = lax.fori_loop(jlo, jhi, jbody, (l, jnp.int32(0)))

            @pl.when(rc > 0)
            def _():
                def fissue(s, _):
                    @pl.when(s < rc)
                    def _():
                        pltpu.async_copy(ring.at[s], out_hbm.at[ids[s]], sf)
                    return 0

                lax.fori_loop(0, KE, fissue, 0)

            return (l, rc)

        def pair_body(i, carry):
            carry = do_block(2 * i, 0, carry)
            carry = do_block(2 * i + 1, 1, carry)
            return carry

        l, rcp = lax.fori_loop(0, (NBMAX + 1) // 2, pair_body,
                               (l_init, jnp.int32(0)))

        def fwait2(s, _):
            @pl.when(s < rcp)
            def _():
                pltpu.make_async_copy(ring.at[s], out_hbm.at[0], sf).wait()
            return 0

        lax.fori_loop(0, KE, fwait2, 0)

        @pl.when(e1 > e0)
        def _():
            pltpu.sync_copy(acc, out_hbm.at[n0 + l])

        for c in range(NCH):
            ring[0, pl.ds(16 * c, 16)] = zvec

        def zbody(z, _):
            @pl.when(jnp.logical_and(n0 + z < N, rd(z) == rd(z + 1)))
            def _():
                pltpu.sync_copy(ring.at[0], out_hbm.at[n0 + z])
            return 0

        lax.fori_loop(0, NPW, zbody, 0)

    return k(Pflat, w, row_indices, row_data, iptr_pad)


def kernel(X, edge_index, edge_weight, edge_attr, row_data, row_indices,
           row_indptr, col_data, col_indices, col_indptr,
           W_s1, b_s1, W_s2, b_s2, W_s3, b_s3,
           W_t0, W_t1, W_t2, W_t3, W_t4, W_t5):
    perm = jnp.arange(3 * UNITS).reshape(UNITS, 3).T.reshape(-1)
    W_s3r = W_s3[perm]
    b_s3r = b_s3[perm]

    w = _run_mlp(edge_attr, edge_weight[:, None], W_s1.T, b_s1[None],
                 W_s2.T, b_s2[None],
                 W_s3r.T, b_s3r[None])

    P = _run_pack(X, W_t0.T, W_t1.T, W_t2.T)

    iptr_pad = jnp.concatenate(
        [row_indptr, jnp.full((IPW + NPW,), E, jnp.int32)])
    ri_pad = jnp.concatenate(
        [row_indices, jnp.zeros((EPAD - E,), jnp.int32)])
    rdt_pad = jnp.concatenate(
        [row_data, jnp.zeros((EPAD - E,), jnp.int32)])
    MP = _run_msg(P.reshape(N, PED), w, ri_pad, rdt_pad,
                  iptr_pad).reshape(N, 10, UNITS)

    return _run_post(X, P, MP, W_t3.T, W_t4.T, W_t5.T)


# speedup vs baseline: 1.7417x; 1.1018x over previous
import functools
import jax
import jax.numpy as jnp
from jax import lax
from jax.experimental import pallas as pl
from jax.experimental.pallas import tpu as pltpu
from jax.experimental.pallas import tpu_sc as plsc

N = 10000
E = 160000
NUM_RBF = 32
UNITS = 128
CUTOFF = 5.0

BE = 640
BN = 200
BNC = 80

NW = 32
NPW = 320
IPW = 328
IPV = 344
KE = 24
EPAD = -(-E // KE) * KE
NBMAX = EPAD // KE
PED = 10 * UNITS
NCH = PED // 16


def _mlp_body(ea_ref, ew_ref, w1_ref, b1_ref, w2_ref, b2_ref, w3_ref, b3_ref,
              out_ref):
    a = ea_ref[...]
    h = jax.nn.silu(jnp.dot(a, w1_ref[...], preferred_element_type=jnp.float32)
                    + b1_ref[...])
    h = jax.nn.silu(jnp.dot(h, w2_ref[...], preferred_element_type=jnp.float32)
                    + b2_ref[...])
    h = jax.nn.silu(jnp.dot(h, w3_ref[...], preferred_element_type=jnp.float32)
                    + b3_ref[...])
    r = ew_ref[...]
    c = 0.5 * (jnp.cos(jnp.pi * r / CUTOFF) + 1.0) * (r < CUTOFF).astype(r.dtype)
    out_ref[...] = h * c


def _run_mlp(ea, ew, W_s1T, b_s1, W_s2T, b_s2, W_s3rT, b_s3r):
    grid = E // BE
    full = lambda shape: pl.BlockSpec(shape, lambda i: (0,) * len(shape))
    return pl.pallas_call(
        _mlp_body,
        grid=(grid,),
        in_specs=[
            pl.BlockSpec((BE, NUM_RBF), lambda i: (i, 0)),
            pl.BlockSpec((BE, 1), lambda i: (i, 0)),
            full((NUM_RBF, UNITS)), full((1, UNITS)),
            full((UNITS, 2 * UNITS)), full((1, 2 * UNITS)),
            full((2 * UNITS, 3 * UNITS)), full((1, 3 * UNITS)),
        ],
        out_specs=pl.BlockSpec((BE, 3 * UNITS), lambda i: (i, 0)),
        out_shape=jax.ShapeDtypeStruct((E, 3 * UNITS), jnp.float32),
    )(ea, ew, W_s1T, b_s1, W_s2T, b_s2, W_s3rT, b_s3r)


def _normed_components(x_ref):
    x = [[x_ref[:, i, j, :] for j in range(3)] for i in range(3)]
    nrm = sum(x[i][j] * x[i][j] for i in range(3) for j in range(3)) + 1.0
    inv = 1.0 / nrm
    xn = [[x[i][j] * inv for j in range(3)] for i in range(3)]
    return xn


def _pack_body(x_ref, w0_ref, w1_ref, w2_ref, out_ref):
    xn = _normed_components(x_ref)
    t = (xn[0][0] + xn[1][1] + xn[2][2]) * (1.0 / 3.0)
    a01 = 0.5 * (xn[0][1] - xn[1][0])
    a02 = 0.5 * (xn[0][2] - xn[2][0])
    a12 = 0.5 * (xn[1][2] - xn[2][1])
    s00 = xn[0][0] - t
    s11 = xn[1][1] - t
    s22 = xn[2][2] - t
    s01 = 0.5 * (xn[0][1] + xn[1][0])
    s02 = 0.5 * (xn[0][2] + xn[2][0])
    s12 = 0.5 * (xn[1][2] + xn[2][1])
    p0 = jnp.dot(t, w0_ref[...], preferred_element_type=jnp.float32)
    pa = jnp.dot(jnp.concatenate([a01, a02, a12], axis=0), w1_ref[...],
                 preferred_element_type=jnp.float32)
    ps = jnp.dot(jnp.concatenate([s00, s01, s02, s11, s12, s22], axis=0),
                 w2_ref[...], preferred_element_type=jnp.float32)
    b = p0.shape[0]
    out_ref[:, 0, :] = p0
    for k in range(3):
        out_ref[:, 1 + k, :] = pa[k * b:(k + 1) * b]
    for k in range(6):
        out_ref[:, 4 + k, :] = ps[k * b:(k + 1) * b]


def _run_pack(X, W_t0T, W_t1T, W_t2T):
    grid = N // BN
    full = lambda shape: pl.BlockSpec(shape, lambda i: (0,) * len(shape))
    return pl.pallas_call(
        _pack_body,
        grid=(grid,),
        in_specs=[
            pl.BlockSpec((BN, 3, 3, UNITS), lambda i: (i, 0, 0, 0)),
            full((UNITS, UNITS)), full((UNITS, UNITS)), full((UNITS, UNITS)),
        ],
        out_specs=pl.BlockSpec((BN, 10, UNITS), lambda i: (i, 0, 0)),
        out_shape=jax.ShapeDtypeStruct((N, 10, UNITS), jnp.float32),
    )(X, W_t0T, W_t1T, W_t2T)


def _tensor_from_rows(p):
    t, a01, a02, a12, s00, s01, s02, s11, s12, s22 = p
    return [
        [t + s00, a01 + s01, a02 + s02],
        [s01 - a01, t + s11, a12 + s12],
        [s02 - a02, s12 - a12, t + s22],
    ]


def _post_body(x_ref, p_ref, mp_ref, w3_ref, w4_ref, w5_ref, out_ref):
    xn = _normed_components(x_ref)
    Y = _tensor_from_rows([p_ref[:, k, :] for k in range(10)])
    M = _tensor_from_rows([mp_ref[:, k, :] for k in range(10)])
    Cm = [[sum(Y[i][j] * M[j][k] + M[i][j] * Y[j][k] for j in range(3))
           for k in range(3)] for i in range(3)]
    nrm = sum(Cm[i][j] * Cm[i][j] for i in range(3) for j in range(3)) + 1.0
    inv = 1.0 / nrm
    c = [[Cm[i][j] * inv for j in range(3)] for i in range(3)]
    t2 = (c[0][0] + c[1][1] + c[2][2]) * (1.0 / 3.0)
    a2 = [0.5 * (c[0][1] - c[1][0]), 0.5 * (c[0][2] - c[2][0]),
          0.5 * (c[1][2] - c[2][1])]
    s2 = [c[0][0] - t2, 0.5 * (c[0][1] + c[1][0]), 0.5 * (c[0][2] + c[2][0]),
          c[1][1] - t2, 0.5 * (c[1][2] + c[2][1]), c[2][2] - t2]
    b = t2.shape[0]
    dt = jnp.dot(t2, w3_ref[...], preferred_element_type=jnp.float32)
    da = jnp.dot(jnp.concatenate(a2, axis=0), w4_ref[...],
                 preferred_element_type=jnp.float32)
    ds = jnp.dot(jnp.concatenate(s2, axis=0), w5_ref[...],
                 preferred_element_type=jnp.float32)
    rows = [dt, da[0:b], da[b:2 * b], da[2 * b:3 * b],
            ds[0:b], ds[b:2 * b], ds[2 * b:3 * b], ds[3 * b:4 * b],
            ds[4 * b:5 * b], ds[5 * b:6 * b]]
    dX = _tensor_from_rows(rows)
    for i in range(3):
        for k in range(3):
            out_ref[:, i, k, :] = (xn[i][k] + dX[i][k]
                                   + sum(dX[i][j] * dX[j][k] for j in range(3)))


def _run_post(X, P, MP, W_t3T, W_t4T, W_t5T):
    grid = N // BNC
    full = lambda shape: pl.BlockSpec(shape, lambda i: (0,) * len(shape))
    return pl.pallas_call(
        _post_body,
        grid=(grid,),
        in_specs=[
            pl.BlockSpec((BNC, 3, 3, UNITS), lambda i: (i, 0, 0, 0)),
            pl.BlockSpec((BNC, 10, UNITS), lambda i: (i, 0, 0)),
            pl.BlockSpec((BNC, 10, UNITS), lambda i: (i, 0, 0)),
            full((UNITS, UNITS)), full((UNITS, UNITS)), full((UNITS, UNITS)),
        ],
        out_specs=pl.BlockSpec((BNC, 3, 3, UNITS), lambda i: (i, 0, 0, 0)),
        out_shape=jax.ShapeDtypeStruct((N, 3, 3, UNITS), jnp.float32),
    )(X, P, MP, W_t3T, W_t4T, W_t5T)


def _sc_mesh():
    return plsc.VectorSubcoreMesh(core_axis_name="c", subcore_axis_name="s")


_W_GROUP = [0, 1, 1, 1, 2, 2, 2, 2, 2, 2]


def _run_msg(Pflat, w, row_indices, row_data, iptr_pad):
    @functools.partial(
        pl.kernel,
        out_type=jax.ShapeDtypeStruct((N, PED), jnp.float32),
        mesh=_sc_mesh(),
        scratch_types=[
            pltpu.VMEM((IPV,), jnp.int32),
            pltpu.VMEM((KE,), jnp.int32),
            pltpu.VMEM((KE,), jnp.int32),
            pltpu.VMEM((KE,), jnp.int32),
            pltpu.VMEM((KE,), jnp.int32),
            pltpu.VMEM((KE, PED), jnp.float32),
            pltpu.VMEM((KE, PED), jnp.float32),
            pltpu.VMEM((KE, 3 * UNITS), jnp.float32),
            pltpu.VMEM((KE, 3 * UNITS), jnp.float32),
            pltpu.VMEM((KE, PED), jnp.float32),
            pltpu.VMEM((PED,), jnp.float32),
            pltpu.SMEM((KE,), jnp.int32),
            pltpu.SMEM((1,), jnp.int32),
            pltpu.SemaphoreType.DMA,
            pltpu.SemaphoreType.DMA,
            pltpu.SemaphoreType.DMA,
            pltpu.SemaphoreType.DMA,
            pltpu.SemaphoreType.DMA,
            pltpu.SemaphoreType.DMA,
            pltpu.SemaphoreType.DMA,
        ],
    )
    def k(p_hbm, w_hbm, ri_hbm, rd_hbm, ip_hbm, out_hbm,
          iptr_v, idx0, idx1, wdx0, wdx1, pb0, pb1, wb0, wb1, ring, acc, ids,
          lsl, si0, si1, sg0, sg1, sw0, sw1, sf):
        wid = lax.axis_index("c") * 16 + lax.axis_index("s")
        n0 = wid * NPW
        pltpu.sync_copy(ip_hbm.at[pl.ds(n0, IPW)], iptr_v.at[pl.ds(0, IPW)])

        def rd(i):
            return iptr_v[pl.ds(i, 16)][0]

        zvec = jnp.zeros((16,), jnp.float32)

        def zero_acc():
            for c in range(NCH):
                acc[pl.ds(16 * c, 16)] = zvec

        zero_acc()

        e0 = rd(0)
        e1 = rd(NPW)
        nb = (e1 + KE - 1) // KE - e0 // KE
        b0 = e0 // KE

        def binsearch(e):
            lo = jnp.int32(0)
            for s in (256, 128, 64, 32, 16, 8, 4, 2, 1):
                cand = lo + s
                ok = jnp.logical_and(cand <= NPW - 1, rd(cand) <= e)
                lo = jnp.where(ok, cand, lo)
            return lo

        l_init = binsearch(e0)

        bufs = ((idx0, wdx0, pb0, wb0, si0, sg0, sw0),
                (idx1, wdx1, pb1, wb1, si1, sg1, sw1))

        def idx_fetch(r, t):
            ix, wx, _, _, si, _, _ = bufs[t]
            b = b0 + r
            pltpu.async_copy(ri_hbm.at[pl.ds(b * KE, KE)], ix, si)
            pltpu.async_copy(rd_hbm.at[pl.ds(b * KE, KE)], wx, si)

        def idx_wait(r, t):
            ix, wx, _, _, si, _, _ = bufs[t]
            b = b0 + r
            pltpu.make_async_copy(ri_hbm.at[pl.ds(b * KE, KE)], ix, si).wait()
            pltpu.make_async_copy(rd_hbm.at[pl.ds(b * KE, KE)], wx, si).wait()

        def data_fetch(t):
            ix, wx, pb, wb, _, sg, sw = bufs[t]
            pltpu.async_copy(p_hbm.at[ix], pb, sg)
            pltpu.async_copy(w_hbm.at[wx], wb, sw)

        def data_wait(t):
            ix, wx, pb, wb, _, sg, sw = bufs[t]
            pltpu.make_async_copy(p_hbm.at[ix], pb, sg).wait()
            pltpu.make_async_copy(w_hbm.at[wx], wb, sw).wait()

        @pl.when(nb > 0)
        def _():
            b = b0
            pltpu.sync_copy(ri_hbm.at[pl.ds(b * KE, KE)], idx0)
            pltpu.sync_copy(rd_hbm.at[pl.ds(b * KE, KE)], wdx0)
            data_fetch(0)

        @pl.when(nb > 1)
        def _():
            idx_fetch(1, 1)

        def do_block(r, t, carry):
            l, rcp = carry
            b = b0 + r

            @pl.when(r + 1 < nb)
            def _():
                idx_wait(r + 1, 1 - t)
                data_fetch(1 - t)

            @pl.when(r < nb)
            def _():
                data_wait(t)

            @pl.when(r + 2 < nb)
            def _():
                idx_fetch(r + 2, t)

            @pl.when(rcp > 0)
            def _():
                def fwait(s, _):
                    @pl.when(s < rcp)
                    def _():
                        pltpu.make_async_copy(ring.at[s], out_hbm.at[0],
                                              sf).wait()
                    return 0

                lax.fori_loop(0, KE, fwait, 0)

            _, _, pb, wb, _, _, _ = bufs[t]
            jlo = jnp.maximum(e0 - b * KE, 0)
            jhi = jnp.minimum(e1 - b * KE, KE)

            def jbody(j, c2):
                l_, rc = c2
                e = b * KE + j
                close = rd(l_ + 1) <= e

                @pl.when(close)
                def _():
                    for c3 in range(NCH):
                        sl = pl.ds(16 * c3, 16)
                        ring[rc, sl] = acc[sl]
                        acc[sl] = zvec
                    ids[rc] = n0 + l_
                    lsl[0] = binsearch(e)

                rc = rc + close.astype(jnp.int32)
                l_ = jnp.where(close, lsl[0], l_)

                wv = [wb[j, pl.ds(128 * g + 16 * m, 16)]
                      for g in range(3) for m in range(8)]
                for k10 in range(10):
                    g = _W_GROUP[k10]
                    for m in range(8):
                        ch = k10 * 8 + m
                        plsc.addupdate(
                            acc.at[pl.ds(16 * ch, 16)],
                            pb[j, pl.ds(16 * ch, 16)] * wv[g * 8 + m])
                return (l_, rc)

            l, rc = lax.fori_loop(jlo, jhi, jbody, (l, jnp.int32(0)))

            @pl.when(rc > 0)
            def _():
                def fissue(s, _):
                    @pl.when(s < rc)
                    def _():
                        pltpu.async_copy(ring.at[s], out_hbm.at[ids[s]], sf)
                    return 0

                lax.fori_loop(0, KE, fissue, 0)

            return (l, rc)

        def pair_body(i, carry):
            carry = do_block(2 * i, 0, carry)
            carry = do_block(2 * i + 1, 1, carry)
            return carry

        l, rcp = lax.fori_loop(0, (NBMAX + 1) // 2, pair_body,
                               (l_init, jnp.int32(0)))

        def fwait2(s, _):
            @pl.when(s < rcp)
            def _():
                pltpu.make_async_copy(ring.at[s], out_hbm.at[0], sf).wait()
            return 0

        lax.fori_loop(0, KE, fwait2, 0)

        @pl.when(e1 > e0)
        def _():
            pltpu.sync_copy(acc, out_hbm.at[n0 + l])

        for c in range(NCH):
            ring[0, pl.ds(16 * c, 16)] = zvec

        def zbody(z, _):
            @pl.when(jnp.logical_and(n0 + z < N, rd(z) == rd(z + 1)))
            def _():
                pltpu.sync_copy(ring.at[0], out_hbm.at[n0 + z])
            return 0

        lax.fori_loop(0, NPW, zbody, 0)

    return k(Pflat, w, row_indices, row_data, iptr_pad)


def kernel(X, edge_index, edge_weight, edge_attr, row_data, row_indices,
           row_indptr, col_data, col_indices, col_indptr,
           W_s1, b_s1, W_s2, b_s2, W_s3, b_s3,
           W_t0, W_t1, W_t2, W_t3, W_t4, W_t5):
    perm = jnp.arange(3 * UNITS).reshape(UNITS, 3).T.reshape(-1)
    W_s3r = W_s3[perm]
    b_s3r = b_s3[perm]

    w = _run_mlp(edge_attr, edge_weight[:, None], W_s1.T, b_s1[None],
                 W_s2.T, b_s2[None],
                 W_s3r.T, b_s3r[None])

    P = _run_pack(X, W_t0.T, W_t1.T, W_t2.T)

    iptr_pad = jnp.concatenate(
        [row_indptr, jnp.full((IPW + NPW,), E, jnp.int32)])
    ri_pad = jnp.concatenate(
        [row_indices, jnp.zeros((EPAD - E,), jnp.int32)])
    rdt_pad = jnp.concatenate(
        [row_data, jnp.zeros((EPAD - E,), jnp.int32)])
    MP = _run_msg(P.reshape(N, PED), w, ri_pad, rdt_pad,
                  iptr_pad).reshape(N, 10, UNITS)

    return _run_post(X, P, MP, W_t3.T, W_t4.T, W_t5.T)
